# baseline (speedup 1.0000x reference)
import jax
import jax.numpy as jnp
from jax import lax
from jax.experimental import pallas as pl
from jax.experimental.pallas import tpu as pltpu
from jax.experimental.pallas import tpu_sc as plsc

NC = 2
NS = 16
L = 16
NW = NC * NS
BATCH = 16384
D = 64
BPW = BATCH // NW
KC = 128
NCHUNK = BPW // KC
GROUPS = BPW // L


def _body(inputs_hbm, user_hbm, item_hbm, out_hbm,
          ids_v, uid_v, iid_v, urows_v, irows_v, out_v, sem):
    wid = lax.axis_index("s") * NC + lax.axis_index("c")
    base = pl.multiple_of(wid * BPW, BPW)

    pltpu.sync_copy(inputs_hbm.at[pl.ds(base * 2, BPW * 2)], ids_v)

    lane = lax.iota(jnp.int32, L)
    lane2 = lane * 2
    for c in range(NCHUNK):
        for g in range(KC // L):
            rows = lane2 + (c * KC + g * L) * 2
            uid_v[c, pl.ds(g * L, L)] = plsc.load_gather(ids_v, [rows])
            iid_v[c, pl.ds(g * L, L)] = plsc.load_gather(ids_v, [rows + 1])

    copies = []
    for c in range(NCHUNK):
        copies.append(pltpu.async_copy(
            user_hbm.at[uid_v.at[c]], urows_v.at[pl.ds(c * KC, KC)], sem))
        copies.append(pltpu.async_copy(
            item_hbm.at[iid_v.at[c]], irows_v.at[pl.ds(c * KC, KC)], sem))
    for cp in copies:
        cp.wait()

    def group(g, carry):
        goff = pl.multiple_of(g * L, L)
        rows = lane + goff
        acc = jnp.zeros((L,), jnp.float32)
        for d in range(D):
            cd = jnp.full((L,), d, jnp.int32)
            uv = plsc.load_gather(urows_v, [rows, cd])
            iv = plsc.load_gather(irows_v, [rows, cd])
            acc = acc + uv * iv
        out_v[pl.ds(goff, L)] = acc
        return carry

    lax.fori_loop(0, GROUPS, group, 0)

    pltpu.sync_copy(out_v, out_hbm.at[pl.ds(base, BPW)])


def kernel(inputs, user_table, item_table):
    mesh = plsc.VectorSubcoreMesh(core_axis_name="c", subcore_axis_name="s",
                                  num_cores=NC, num_subcores=NS)
    f = pl.kernel(
        _body,
        out_type=jax.ShapeDtypeStruct((BATCH,), jnp.float32),
        mesh=mesh,
        compiler_params=pltpu.CompilerParams(needs_layout_passes=False,
                                             use_tc_tiling_on_sc=False),
        scratch_types=[
            pltpu.VMEM((BPW * 2,), jnp.int32),
            pltpu.VMEM((NCHUNK, KC), jnp.int32),
            pltpu.VMEM((NCHUNK, KC), jnp.int32),
            pltpu.VMEM((BPW, D), jnp.float32),
            pltpu.VMEM((BPW, D), jnp.float32),
            pltpu.VMEM((BPW,), jnp.float32),
            pltpu.SemaphoreType.DMA,
        ],
    )
    return f(inputs.reshape(-1), user_table, item_table)


# speedup vs baseline: 1.2054x; 1.2054x over previous
import jax
import jax.numpy as jnp
from jax import lax
from jax.experimental import pallas as pl
from jax.experimental.pallas import tpu as pltpu
from jax.experimental.pallas import tpu_sc as plsc

NC = 2
NS = 16
L = 16
NW = NC * NS
BATCH = 16384
D = 64
W = 2 * D
BPW = BATCH // NW
KC = 128
NCHUNK = BPW // KC
GPC = KC // L


def _body(ids_hbm, big_hbm, out_hbm,
          ids_v, uid_v, iid_v, urows_v, irows_v, out_v, sem_u, sem_i):
    wid = lax.axis_index("s") * NC + lax.axis_index("c")
    base = pl.multiple_of(wid * BPW, BPW)

    pltpu.sync_copy(ids_hbm.at[pl.ds(base * 2, BPW * 2)], ids_v)

    lane = lax.iota(jnp.int32, L)
    lane2 = lane * 2
    for c in range(NCHUNK):
        for g in range(GPC):
            rows = lane2 + (c * KC + g * L) * 2
            uid_v[c, pl.ds(g * L, L)] = plsc.load_gather(ids_v, [rows])
            iid_v[c, pl.ds(g * L, L)] = plsc.load_gather(ids_v, [rows + 1])

    def compute_group(c, g):
        goff = pl.multiple_of(g * L, L)
        rows = lane + goff
        ub = urows_v.at[c % 2]
        ib = irows_v.at[c % 2]
        acc = jnp.zeros((L,), jnp.float32)
        for d in range(D):
            cu = jnp.full((L,), d, jnp.int32)
            ci = jnp.full((L,), D + d, jnp.int32)
            acc = acc + (plsc.load_gather(ub, [rows, cu]) *
                         plsc.load_gather(ib, [rows, ci]))
        out_v[pl.ds(c * KC + goff, L)] = acc

    def fire(c):
        cp_u = pltpu.async_copy(big_hbm.at[uid_v.at[c]], urows_v.at[c % 2],
                                sem_u)
        cp_i = pltpu.async_copy(big_hbm.at[iid_v.at[c]], irows_v.at[c % 2],
                                sem_i)
        return cp_u, cp_i

    pending = fire(0)
    for c in range(NCHUNK):
        pending[0].wait()
        pending[1].wait()
        if c + 1 < NCHUNK:
            pending = fire(c + 1)

        def group_body(g, carry):
            compute_group(c, g)
            return carry

        lax.fori_loop(0, GPC, group_body, 0)

    pltpu.sync_copy(out_v, out_hbm.at[pl.ds(base, BPW)])


def kernel(inputs, user_table, item_table):
    big = jnp.concatenate([user_table, item_table], axis=1)
    mesh = plsc.VectorSubcoreMesh(core_axis_name="c", subcore_axis_name="s",
                                  num_cores=NC, num_subcores=NS)
    f = pl.kernel(
        _body,
        out_type=jax.ShapeDtypeStruct((BATCH,), jnp.float32),
        mesh=mesh,
        compiler_params=pltpu.CompilerParams(needs_layout_passes=False),
        scratch_types=[
            pltpu.VMEM((BPW * 2,), jnp.int32),
            pltpu.VMEM((NCHUNK, KC), jnp.int32),
            pltpu.VMEM((NCHUNK, KC), jnp.int32),
            pltpu.VMEM((2, KC, W), jnp.float32),
            pltpu.VMEM((2, KC, W), jnp.float32),
            pltpu.VMEM((BPW,), jnp.float32),
            pltpu.SemaphoreType.DMA,
            pltpu.SemaphoreType.DMA,
        ],
    )
    return f(inputs.reshape(-1), big)
